# baseline (speedup 1.0000x reference)
import jax
import jax.numpy as jnp
from jax.experimental import pallas as pl
from jax.experimental.pallas import tpu as pltpu

ROWS = 16 * 2048
COLS = 512
BLK = 1024
GRID = ROWS // BLK
H = BLK // 4

_LN2_126 = 126.0 * 0.6931471805599453
_SCALE = 2.0 ** 126


def _loss_kernel(hard_ref, soft_ref, out_ref, acc_ref, cnt_ref):
    i = pl.program_id(0)

    @pl.when(i == 0)
    def _init():
        acc_ref[...] = jnp.zeros_like(acc_ref)
        cnt_ref[...] = jnp.zeros_like(cnt_ref)

    zero = jnp.zeros((8, COLS), jnp.float32)
    zeroi = jnp.zeros((8, COLS), jnp.int32)
    accs = [zero]
    cnts = [zeroi]
    for k in range(BLK // 32):
        xs, hs = [], []
        for q in range(4):
            sl = pl.ds(k * 32 + q * 8, 8)
            h = hard_ref[sl]
            xs.append(jnp.where(h == 1, jnp.maximum(soft_ref[sl], 1e-12), 1.0))
            hs.append(h)
        p = ((xs[0] * xs[1]) * _SCALE) * (xs[2] * xs[3])
        accs[0] += jnp.log(p) - _LN2_126
        cnts[0] += (hs[0] + hs[1]) + (hs[2] + hs[3])

    acc_ref[...] += accs[0]
    cnt_ref[...] += cnts[0].astype(jnp.float32)

    @pl.when(i == GRID - 1)
    def _fini():
        total = jnp.sum(acc_ref[...])
        count = jnp.sum(cnt_ref[...])
        out_ref[0, 0] = -total / count


def kernel(hard_attention, soft_attention):
    hard = hard_attention.reshape(ROWS, COLS)
    soft = soft_attention.reshape(ROWS, COLS)
    out = pl.pallas_call(
        _loss_kernel,
        grid=(GRID,),
        in_specs=[
            pl.BlockSpec((BLK, COLS), lambda i: (i, 0)),
            pl.BlockSpec((BLK, COLS), lambda i: (i, 0)),
        ],
        out_specs=pl.BlockSpec(memory_space=pltpu.SMEM),
        out_shape=jax.ShapeDtypeStruct((1, 1), jnp.float32),
        scratch_shapes=[
            pltpu.VMEM((8, COLS), jnp.float32),
            pltpu.VMEM((8, COLS), jnp.float32),
        ],
        compiler_params=pltpu.CompilerParams(
            dimension_semantics=("arbitrary",),
        ),
    )(hard, soft)
    return out[0, 0]


# speedup vs baseline: 1.1433x; 1.1433x over previous
import jax
import jax.numpy as jnp
from jax.experimental import pallas as pl
from jax.experimental.pallas import tpu as pltpu

ROWS = 16 * 2048
COLS = 512
BLK = 2048
GRID = ROWS // BLK
H = BLK // 4

_LN2_126 = 126.0 * 0.6931471805599453
_SCALE = 2.0 ** 126


def _loss_kernel(hard_ref, soft_ref, out_ref, acc_ref, cnt_ref):
    i = pl.program_id(0)

    @pl.when(i == 0)
    def _init():
        acc_ref[...] = jnp.zeros_like(acc_ref)
        cnt_ref[...] = jnp.zeros_like(cnt_ref)

    zero = jnp.zeros((8, COLS), jnp.float32)
    zeroi = jnp.zeros((8, COLS), jnp.int32)
    accs = [zero]
    cnts = [zeroi]
    for k in range(BLK // 32):
        xs, hs = [], []
        for q in range(4):
            sl = pl.ds(k * 32 + q * 8, 8)
            h = hard_ref[sl]
            xs.append(jnp.where(h == 1, jnp.maximum(soft_ref[sl], 1e-12), 1.0))
            hs.append(h)
        p = ((xs[0] * xs[1]) * _SCALE) * (xs[2] * xs[3])
        accs[0] += jnp.log(p) - _LN2_126
        cnts[0] += (hs[0] + hs[1]) + (hs[2] + hs[3])

    acc_ref[...] += accs[0]
    cnt_ref[...] += cnts[0].astype(jnp.float32)

    @pl.when(i == GRID - 1)
    def _fini():
        total = jnp.sum(acc_ref[...])
        count = jnp.sum(cnt_ref[...])
        out_ref[0, 0] = -total / count


def kernel(hard_attention, soft_attention):
    hard = hard_attention.reshape(ROWS, COLS)
    soft = soft_attention.reshape(ROWS, COLS)
    out = pl.pallas_call(
        _loss_kernel,
        grid=(GRID,),
        in_specs=[
            pl.BlockSpec((BLK, COLS), lambda i: (i, 0)),
            pl.BlockSpec((BLK, COLS), lambda i: (i, 0)),
        ],
        out_specs=pl.BlockSpec(memory_space=pltpu.SMEM),
        out_shape=jax.ShapeDtypeStruct((1, 1), jnp.float32),
        scratch_shapes=[
            pltpu.VMEM((8, COLS), jnp.float32),
            pltpu.VMEM((8, COLS), jnp.float32),
        ],
        compiler_params=pltpu.CompilerParams(
            dimension_semantics=("arbitrary",),
        ),
    )(hard, soft)
    return out[0, 0]
